# baseline (speedup 1.0000x reference)
import functools

import jax
import jax.numpy as jnp
from jax import lax
from jax.experimental import pallas as pl
from jax.experimental.pallas import tpu as pltpu
from jax.experimental.pallas import tpu_sc as plsc

N = 10000
E = 320000
D = 128
H = 128
R = 4
T = 2
L = 2

NC = 2
NS = 16
NW = NC * NS

K = 128
CHUNKS = 80
E_PAD = NW * CHUNKS * K
N_ACC = 10112
ROWS_PER_TILE = N_ACC // NS


def _msg_body(h_ref, w1_ref, b1_ref, w2_ref, b2_ref, out_ref):
    r = pl.program_id(0)
    h = h_ref[...]
    m1 = jnp.maximum(
        jnp.dot(h, w1_ref[0], preferred_element_type=jnp.float32) + b1_ref[r][None, :],
        0.0,
    )
    out_ref[0] = jnp.dot(m1, w2_ref[0], preferred_element_type=jnp.float32) + b2_ref[r][None, :]


def _messages(h, We1, be1, We2, be2, bn):
    nb = N // bn
    return pl.pallas_call(
        _msg_body,
        grid=(R, nb),
        in_specs=[
            pl.BlockSpec((bn, D), lambda r, i: (i, 0)),
            pl.BlockSpec((1, D, H), lambda r, i: (r, 0, 0)),
            pl.BlockSpec((R, H), lambda r, i: (0, 0)),
            pl.BlockSpec((1, H, D), lambda r, i: (r, 0, 0)),
            pl.BlockSpec((R, D), lambda r, i: (0, 0)),
        ],
        out_specs=pl.BlockSpec((1, bn, D), lambda r, i: (r, i, 0)),
        out_shape=jax.ShapeDtypeStruct((R, N, D), jnp.float32),
    )(h, We1, be1, We2, be2)


def _rowidx_body(src_ref, et_ref, out_ref):
    out_ref[...] = et_ref[...] * N + src_ref[...]


def _row_indices(src2d, et2d):
    rows = src2d.shape[0]
    br = rows // 5
    return pl.pallas_call(
        _rowidx_body,
        grid=(rows // br,),
        in_specs=[
            pl.BlockSpec((br, K), lambda i: (i, 0)),
            pl.BlockSpec((br, K), lambda i: (i, 0)),
        ],
        out_specs=pl.BlockSpec((br, K), lambda i: (i, 0)),
        out_shape=jax.ShapeDtypeStruct((rows, K), jnp.int32),
    )(src2d, et2d)


def _sc_scatter_body(m_hbm, rowidx_hbm, dst_hbm, zeros_hbm, out_hbm,
                     idx_v, dstv, rows0, rows1, agg_sh, sem0, sem1, ssem0, ssem1):
    c = lax.axis_index("c")
    s = lax.axis_index("s")
    gw = s * NC + c

    r0 = s * ROWS_PER_TILE
    pltpu.sync_copy(zeros_hbm.at[pl.ds(r0, ROWS_PER_TILE)],
                    agg_sh.at[pl.ds(r0, ROWS_PER_TILE)])

    plsc.subcore_barrier()

    half = CHUNKS // 2
    for g in range(2):
        pltpu.sync_copy(rowidx_hbm.at[gw].at[pl.ds(g * half, half)], idx_v)
        pltpu.sync_copy(dst_hbm.at[gw].at[pl.ds(g * half, half)], dstv)
        pltpu.async_copy(m_hbm.at[idx_v.at[0]], rows0, sem0)
        pltpu.async_copy(m_hbm.at[idx_v.at[1]], rows1, sem1)

        def body(i, carry):
            j = 2 * i
            pltpu.make_async_copy(m_hbm.at[idx_v.at[j]], rows0, sem0).wait()
            s0 = pltpu.async_copy(rows0, agg_sh.at[dstv.at[j]], ssem0, add=True)
            pltpu.make_async_copy(m_hbm.at[idx_v.at[j + 1]], rows1, sem1).wait()
            s1 = pltpu.async_copy(rows1, agg_sh.at[dstv.at[j + 1]], ssem1, add=True)
            s0.wait()

            @pl.when(j + 2 < half)
            def _():
                pltpu.async_copy(m_hbm.at[idx_v.at[j + 2]], rows0, sem0)

            s1.wait()

            @pl.when(j + 3 < half)
            def _():
                pltpu.async_copy(m_hbm.at[idx_v.at[j + 3]], rows1, sem1)

            return carry

        lax.fori_loop(0, half // 2, body, 0)
    plsc.subcore_barrier()

    pltpu.sync_copy(agg_sh.at[pl.ds(r0, ROWS_PER_TILE)],
                    out_hbm.at[c].at[pl.ds(r0, ROWS_PER_TILE)])


@functools.lru_cache(maxsize=1)
def _sc_scatter_kernel():
    return pl.kernel(
        _sc_scatter_body,
        mesh=plsc.VectorSubcoreMesh(
            core_axis_name="c", subcore_axis_name="s",
            num_cores=NC, num_subcores=NS,
        ),
        out_type=jax.ShapeDtypeStruct((NC, N_ACC, D), jnp.float32),
        scratch_types=[
            pltpu.VMEM((CHUNKS // 2, K), jnp.int32),
            pltpu.VMEM((CHUNKS // 2, K), jnp.int32),
            pltpu.VMEM((K, D), jnp.float32),
            pltpu.VMEM((K, D), jnp.float32),
            pltpu.VMEM_SHARED((N_ACC, D), jnp.float32),
            pltpu.SemaphoreType.DMA,
            pltpu.SemaphoreType.DMA,
            pltpu.SemaphoreType.DMA,
            pltpu.SemaphoreType.DMA,
        ],
    )


def _update_body(h_ref, parts_ref, nt_ref, w1_ref, b1_ref, w2_ref, b2_ref, out_ref):
    h = h_ref[...]
    agg = parts_ref[0] + parts_ref[1]
    z = jnp.concatenate([h, agg], axis=1)
    nt = nt_ref[...]
    acc = h
    for t in range(T):
        u1 = jnp.maximum(
            jnp.dot(z, w1_ref[t], preferred_element_type=jnp.float32) + b1_ref[t][None, :],
            0.0,
        )
        u = jnp.dot(u1, w2_ref[t], preferred_element_type=jnp.float32) + b2_ref[t][None, :]
        acc = jnp.where(nt == t, u, acc)
    out_ref[...] = acc


def _update(h, parts, nt3, Wn1, bn1, Wn2, bn2, bn):
    nb = N // bn
    return pl.pallas_call(
        _update_body,
        grid=(nb,),
        in_specs=[
            pl.BlockSpec((bn, D), lambda i: (i, 0)),
            pl.BlockSpec((NC, bn, D), lambda i: (0, i, 0)),
            pl.BlockSpec((bn, 1), lambda i: (i, 0)),
            pl.BlockSpec((T, 2 * D, H), lambda i: (0, 0, 0)),
            pl.BlockSpec((T, H), lambda i: (0, 0)),
            pl.BlockSpec((T, H, D), lambda i: (0, 0, 0)),
            pl.BlockSpec((T, D), lambda i: (0, 0)),
        ],
        out_specs=pl.BlockSpec((bn, D), lambda i: (i, 0)),
        out_shape=jax.ShapeDtypeStruct((N, D), jnp.float32),
    )(h, parts, nt3, Wn1, bn1, Wn2, bn2)


def kernel(node_feature, edge_index, edge_types, node_types,
           update_node_type_indices, update_edge_type_indices,
           We1, be1, We2, be2, Wn1, bn1, Wn2, bn2):
    del update_node_type_indices, update_edge_type_indices

    src = edge_index[0].astype(jnp.int32)
    dst = edge_index[1].astype(jnp.int32)
    et = edge_types.astype(jnp.int32)

    pad = E_PAD - E
    pad_iota = lax.iota(jnp.int32, pad)
    src2d = jnp.concatenate([src, pad_iota % N]).reshape(E_PAD // K, K)
    et2d = jnp.concatenate([et, jnp.zeros((pad,), jnp.int32)]).reshape(E_PAD // K, K)
    dst3 = jnp.concatenate([dst, N + pad_iota % (N_ACC - N)]).reshape(NW, CHUNKS, K)

    rowidx3 = _row_indices(src2d, et2d).reshape(NW, CHUNKS, K)

    zeros_acc = jnp.zeros((N_ACC, D), jnp.float32)
    nt3 = node_types.astype(jnp.int32).reshape(N, 1)

    bn = 1000
    h = node_feature
    for l in range(L):
        m = _messages(h, We1[l], be1[l], We2[l], be2[l], bn=2000)
        m_flat = m.reshape(R * N, D)
        parts = _sc_scatter_kernel()(m_flat, rowidx3, dst3, zeros_acc)
        h = _update(h, parts, nt3, Wn1[l], bn1[l], Wn2[l], bn2[l], bn=bn)
    return h


# speedup vs baseline: 1.1962x; 1.1962x over previous
import functools

import jax
import jax.numpy as jnp
from jax import lax
from jax.experimental import pallas as pl
from jax.experimental.pallas import tpu as pltpu
from jax.experimental.pallas import tpu_sc as plsc

N = 10000
E = 320000
D = 128
H = 128
R = 4
T = 2
L = 2

NC = 2
NS = 16
NW = NC * NS

K = 128
CHUNKS = 80
E_PAD = NW * CHUNKS * K
N_ACC = 10112
ROWS_PER_TILE = N_ACC // NS


def _msg_body(h_ref, w1_ref, b1_ref, w2_ref, b2_ref, out_ref):
    r = pl.program_id(0)
    h = h_ref[...].astype(jnp.bfloat16)
    m1 = jnp.maximum(
        jnp.dot(h, w1_ref[0].astype(jnp.bfloat16),
                preferred_element_type=jnp.float32) + b1_ref[r][None, :],
        0.0,
    ).astype(jnp.bfloat16)
    out_ref[0] = jnp.dot(m1, w2_ref[0].astype(jnp.bfloat16),
                         preferred_element_type=jnp.float32) + b2_ref[r][None, :]


def _messages(h, We1, be1, We2, be2, bn):
    nb = N // bn
    return pl.pallas_call(
        _msg_body,
        grid=(R, nb),
        in_specs=[
            pl.BlockSpec((bn, D), lambda r, i: (i, 0)),
            pl.BlockSpec((1, D, H), lambda r, i: (r, 0, 0)),
            pl.BlockSpec((R, H), lambda r, i: (0, 0)),
            pl.BlockSpec((1, H, D), lambda r, i: (r, 0, 0)),
            pl.BlockSpec((R, D), lambda r, i: (0, 0)),
        ],
        out_specs=pl.BlockSpec((1, bn, D), lambda r, i: (r, i, 0)),
        out_shape=jax.ShapeDtypeStruct((R, N, D), jnp.float32),
    )(h, We1, be1, We2, be2)


def _rowidx_body(src_ref, et_ref, out_ref):
    out_ref[...] = et_ref[...] * N + src_ref[...]


def _row_indices(src2d, et2d):
    rows = src2d.shape[0]
    br = rows // 5
    return pl.pallas_call(
        _rowidx_body,
        grid=(rows // br,),
        in_specs=[
            pl.BlockSpec((br, K), lambda i: (i, 0)),
            pl.BlockSpec((br, K), lambda i: (i, 0)),
        ],
        out_specs=pl.BlockSpec((br, K), lambda i: (i, 0)),
        out_shape=jax.ShapeDtypeStruct((rows, K), jnp.int32),
    )(src2d, et2d)


def _sc_scatter_body(m_hbm, rowidx_hbm, dst_hbm, zeros_hbm, out_hbm,
                     idx_v, dstv, rows0, rows1, agg_sh, sem0, sem1):
    c = lax.axis_index("c")
    s = lax.axis_index("s")
    gw = s * NC + c

    r0 = s * ROWS_PER_TILE
    pltpu.sync_copy(zeros_hbm.at[pl.ds(r0, ROWS_PER_TILE)],
                    agg_sh.at[pl.ds(r0, ROWS_PER_TILE)])

    plsc.subcore_barrier()

    half = CHUNKS // 2
    for g in range(2):
        pltpu.sync_copy(rowidx_hbm.at[gw].at[pl.ds(g * half, half)], idx_v)
        pltpu.sync_copy(dst_hbm.at[gw].at[pl.ds(g * half, half)], dstv)
        pltpu.async_copy(m_hbm.at[idx_v.at[0]], rows0, sem0)

        def body(i, carry):
            j = 2 * i
            pltpu.async_copy(m_hbm.at[idx_v.at[j + 1]], rows1, sem1)
            pltpu.make_async_copy(m_hbm.at[idx_v.at[j]], rows0, sem0).wait()
            pltpu.sync_copy(rows0, agg_sh.at[dstv.at[j]], add=True)

            @pl.when(j + 2 < half)
            def _():
                pltpu.async_copy(m_hbm.at[idx_v.at[j + 2]], rows0, sem0)

            pltpu.make_async_copy(m_hbm.at[idx_v.at[j + 1]], rows1, sem1).wait()
            pltpu.sync_copy(rows1, agg_sh.at[dstv.at[j + 1]], add=True)
            return carry

        lax.fori_loop(0, half // 2, body, 0)
    plsc.subcore_barrier()

    pltpu.sync_copy(agg_sh.at[pl.ds(r0, ROWS_PER_TILE)],
                    out_hbm.at[c].at[pl.ds(r0, ROWS_PER_TILE)])


@functools.lru_cache(maxsize=1)
def _sc_scatter_kernel():
    return pl.kernel(
        _sc_scatter_body,
        mesh=plsc.VectorSubcoreMesh(
            core_axis_name="c", subcore_axis_name="s",
            num_cores=NC, num_subcores=NS,
        ),
        out_type=jax.ShapeDtypeStruct((NC, N_ACC, D), jnp.float32),
        scratch_types=[
            pltpu.VMEM((CHUNKS // 2, K), jnp.int32),
            pltpu.VMEM((CHUNKS // 2, K), jnp.int32),
            pltpu.VMEM((K, D), jnp.float32),
            pltpu.VMEM((K, D), jnp.float32),
            pltpu.VMEM_SHARED((N_ACC, D), jnp.float32),
            pltpu.SemaphoreType.DMA,
            pltpu.SemaphoreType.DMA,
        ],
    )


def _update_body(h_ref, parts_ref, nt_ref, w1_ref, b1_ref, w2_ref, b2_ref, out_ref):
    h = h_ref[...]
    agg = parts_ref[0] + parts_ref[1]
    z = jnp.concatenate([h, agg], axis=1).astype(jnp.bfloat16)
    nt = nt_ref[...]
    acc = h
    for t in range(T):
        u1 = jnp.maximum(
            jnp.dot(z, w1_ref[t].astype(jnp.bfloat16),
                    preferred_element_type=jnp.float32) + b1_ref[t][None, :],
            0.0,
        ).astype(jnp.bfloat16)
        u = jnp.dot(u1, w2_ref[t].astype(jnp.bfloat16),
                    preferred_element_type=jnp.float32) + b2_ref[t][None, :]
        acc = jnp.where(nt == t, u, acc)
    out_ref[...] = acc


def _update(h, parts, nt3, Wn1, bn1, Wn2, bn2, bn):
    nb = N // bn
    return pl.pallas_call(
        _update_body,
        grid=(nb,),
        in_specs=[
            pl.BlockSpec((bn, D), lambda i: (i, 0)),
            pl.BlockSpec((NC, bn, D), lambda i: (0, i, 0)),
            pl.BlockSpec((bn, 1), lambda i: (i, 0)),
            pl.BlockSpec((T, 2 * D, H), lambda i: (0, 0, 0)),
            pl.BlockSpec((T, H), lambda i: (0, 0)),
            pl.BlockSpec((T, H, D), lambda i: (0, 0, 0)),
            pl.BlockSpec((T, D), lambda i: (0, 0)),
        ],
        out_specs=pl.BlockSpec((bn, D), lambda i: (i, 0)),
        out_shape=jax.ShapeDtypeStruct((N, D), jnp.float32),
    )(h, parts, nt3, Wn1, bn1, Wn2, bn2)


def kernel(node_feature, edge_index, edge_types, node_types,
           update_node_type_indices, update_edge_type_indices,
           We1, be1, We2, be2, Wn1, bn1, Wn2, bn2):
    del update_node_type_indices, update_edge_type_indices

    src = edge_index[0].astype(jnp.int32)
    dst = edge_index[1].astype(jnp.int32)
    et = edge_types.astype(jnp.int32)

    pad = E_PAD - E
    pad_iota = lax.iota(jnp.int32, pad)
    src2d = jnp.concatenate([src, pad_iota % N]).reshape(E_PAD // K, K)
    et2d = jnp.concatenate([et, jnp.zeros((pad,), jnp.int32)]).reshape(E_PAD // K, K)
    dst3 = jnp.concatenate([dst, N + pad_iota % (N_ACC - N)]).reshape(NW, CHUNKS, K)

    rowidx3 = _row_indices(src2d, et2d).reshape(NW, CHUNKS, K)

    zeros_acc = jnp.zeros((N_ACC, D), jnp.float32)
    nt3 = node_types.astype(jnp.int32).reshape(N, 1)

    bn = 1000
    h = node_feature
    for l in range(L):
        m = _messages(h, We1[l], be1[l], We2[l], be2[l], bn=2000)
        m_flat = m.reshape(R * N, D)
        parts = _sc_scatter_kernel()(m_flat, rowidx3, dst3, zeros_acc)
        h = _update(h, parts, nt3, Wn1[l], bn1[l], Wn2[l], bn2[l], bn=bn)
    return h


# speedup vs baseline: 1.3001x; 1.0868x over previous
import functools

import jax
import jax.numpy as jnp
from jax import lax
from jax.experimental import pallas as pl
from jax.experimental.pallas import tpu as pltpu
from jax.experimental.pallas import tpu_sc as plsc

N = 10000
E = 320000
D = 128
H = 128
R = 4
T = 2
L = 2

NC = 2
NS = 16
NW = NC * NS

K = 128
CHUNKS = 80
E_PAD = NW * CHUNKS * K
N_ACC = 10112
ROWS_PER_TILE = N_ACC // NS


def _msg_body(h_ref, w1_ref, b1_ref, w2_ref, b2_ref, out_ref):
    h = h_ref[...]
    for r in range(R):
        m1 = jnp.maximum(
            jnp.dot(h, w1_ref[r], preferred_element_type=jnp.float32)
            + b1_ref[r][None, :],
            0.0,
        )
        out_ref[r] = (jnp.dot(m1, w2_ref[r], preferred_element_type=jnp.float32)
                      + b2_ref[r][None, :])


def _messages(h, We1, be1, We2, be2, bn):
    nb = N // bn
    return pl.pallas_call(
        _msg_body,
        grid=(nb,),
        in_specs=[
            pl.BlockSpec((bn, D), lambda i: (i, 0)),
            pl.BlockSpec((R, D, H), lambda i: (0, 0, 0)),
            pl.BlockSpec((R, H), lambda i: (0, 0)),
            pl.BlockSpec((R, H, D), lambda i: (0, 0, 0)),
            pl.BlockSpec((R, D), lambda i: (0, 0)),
        ],
        out_specs=pl.BlockSpec((R, bn, D), lambda i: (0, i, 0)),
        out_shape=jax.ShapeDtypeStruct((R, N, D), jnp.float32),
    )(h, We1, be1, We2, be2)


_PREP_BR = 512


def _prep_body(src_ref, et_ref, dst_ref, ri_ref, dstpad_ref):
    i = pl.program_id(0)
    rows = lax.broadcasted_iota(jnp.int32, (_PREP_BR, K), 0)
    cols = lax.broadcasted_iota(jnp.int32, (_PREP_BR, K), 1)
    flat = (i * _PREP_BR + rows) * K + cols
    mask = flat < E
    ri_ref[...] = jnp.where(mask, et_ref[...] * N + src_ref[...], flat % N)
    dstpad_ref[...] = jnp.where(mask, dst_ref[...], N + flat % (N_ACC - N))


def _edge_prep(src2d, et2d, dst2d):
    rows_out = E_PAD // K
    grid = rows_out // _PREP_BR
    return pl.pallas_call(
        _prep_body,
        grid=(grid,),
        in_specs=[
            pl.BlockSpec((_PREP_BR, K), lambda i: (i, 0)),
            pl.BlockSpec((_PREP_BR, K), lambda i: (i, 0)),
            pl.BlockSpec((_PREP_BR, K), lambda i: (i, 0)),
        ],
        out_specs=[
            pl.BlockSpec((_PREP_BR, K), lambda i: (i, 0)),
            pl.BlockSpec((_PREP_BR, K), lambda i: (i, 0)),
        ],
        out_shape=[
            jax.ShapeDtypeStruct((rows_out, K), jnp.int32),
            jax.ShapeDtypeStruct((rows_out, K), jnp.int32),
        ],
    )(src2d, et2d, dst2d)


def _sc_scatter_body(m_hbm, rowidx_hbm, dst_hbm, zeros_hbm, out_hbm,
                     idx_v, dstv, rows0, rows1, agg_sh, sem0, sem1):
    c = lax.axis_index("c")
    s = lax.axis_index("s")
    gw = s * NC + c

    r0 = s * ROWS_PER_TILE
    pltpu.sync_copy(zeros_hbm.at[pl.ds(r0, ROWS_PER_TILE)],
                    agg_sh.at[pl.ds(r0, ROWS_PER_TILE)])

    plsc.subcore_barrier()

    half = CHUNKS // 2
    for g in range(2):
        pltpu.sync_copy(rowidx_hbm.at[gw].at[pl.ds(g * half, half)], idx_v)
        pltpu.sync_copy(dst_hbm.at[gw].at[pl.ds(g * half, half)], dstv)
        pltpu.async_copy(m_hbm.at[idx_v.at[0]], rows0, sem0)

        def body(i, carry):
            j = 2 * i
            pltpu.async_copy(m_hbm.at[idx_v.at[j + 1]], rows1, sem1)
            pltpu.make_async_copy(m_hbm.at[idx_v.at[j]], rows0, sem0).wait()
            pltpu.sync_copy(rows0, agg_sh.at[dstv.at[j]], add=True)

            @pl.when(j + 2 < half)
            def _():
                pltpu.async_copy(m_hbm.at[idx_v.at[j + 2]], rows0, sem0)

            pltpu.make_async_copy(m_hbm.at[idx_v.at[j + 1]], rows1, sem1).wait()
            pltpu.sync_copy(rows1, agg_sh.at[dstv.at[j + 1]], add=True)
            return carry

        lax.fori_loop(0, half // 2, body, 0)
    plsc.subcore_barrier()

    pltpu.sync_copy(agg_sh.at[pl.ds(r0, ROWS_PER_TILE)],
                    out_hbm.at[c].at[pl.ds(r0, ROWS_PER_TILE)])


@functools.lru_cache(maxsize=1)
def _sc_scatter_kernel():
    return pl.kernel(
        _sc_scatter_body,
        mesh=plsc.VectorSubcoreMesh(
            core_axis_name="c", subcore_axis_name="s",
            num_cores=NC, num_subcores=NS,
        ),
        out_type=jax.ShapeDtypeStruct((NC, N_ACC, D), jnp.float32),
        scratch_types=[
            pltpu.VMEM((CHUNKS // 2, K), jnp.int32),
            pltpu.VMEM((CHUNKS // 2, K), jnp.int32),
            pltpu.VMEM((K, D), jnp.float32),
            pltpu.VMEM((K, D), jnp.float32),
            pltpu.VMEM_SHARED((N_ACC, D), jnp.float32),
            pltpu.SemaphoreType.DMA,
            pltpu.SemaphoreType.DMA,
        ],
    )


def _update_body(h_ref, parts_ref, nt_ref, w1_ref, b1_ref, w2_ref, b2_ref, out_ref):
    h = h_ref[...]
    agg = parts_ref[0] + parts_ref[1]
    z = jnp.concatenate([h, agg], axis=1)
    nt = nt_ref[...]
    acc = h
    for t in range(T):
        u1 = jnp.maximum(
            jnp.dot(z, w1_ref[t], preferred_element_type=jnp.float32) + b1_ref[t][None, :],
            0.0,
        )
        u = jnp.dot(u1, w2_ref[t], preferred_element_type=jnp.float32) + b2_ref[t][None, :]
        acc = jnp.where(nt == t, u, acc)
    out_ref[...] = acc


def _update(h, parts, nt3, Wn1, bn1, Wn2, bn2, bn):
    nb = N // bn
    return pl.pallas_call(
        _update_body,
        grid=(nb,),
        in_specs=[
            pl.BlockSpec((bn, D), lambda i: (i, 0)),
            pl.BlockSpec((NC, bn, D), lambda i: (0, i, 0)),
            pl.BlockSpec((bn, 1), lambda i: (i, 0)),
            pl.BlockSpec((T, 2 * D, H), lambda i: (0, 0, 0)),
            pl.BlockSpec((T, H), lambda i: (0, 0)),
            pl.BlockSpec((T, H, D), lambda i: (0, 0, 0)),
            pl.BlockSpec((T, D), lambda i: (0, 0)),
        ],
        out_specs=pl.BlockSpec((bn, D), lambda i: (i, 0)),
        out_shape=jax.ShapeDtypeStruct((N, D), jnp.float32),
    )(h, parts, nt3, Wn1, bn1, Wn2, bn2)


def kernel(node_feature, edge_index, edge_types, node_types,
           update_node_type_indices, update_edge_type_indices,
           We1, be1, We2, be2, Wn1, bn1, Wn2, bn2):
    del update_node_type_indices, update_edge_type_indices

    src = edge_index[0].astype(jnp.int32)
    dst = edge_index[1].astype(jnp.int32)
    et = edge_types.astype(jnp.int32)

    ri2d, dstpad2d = _edge_prep(src.reshape(E // K, K), et.reshape(E // K, K),
                                dst.reshape(E // K, K))
    rowidx3 = ri2d.reshape(NW, CHUNKS, K)
    dst3 = dstpad2d.reshape(NW, CHUNKS, K)

    zeros_acc = jnp.zeros((N_ACC, D), jnp.float32)
    nt3 = node_types.astype(jnp.int32).reshape(N, 1)

    bn = 1000
    h = node_feature
    for l in range(L):
        m = _messages(h, We1[l], be1[l], We2[l], be2[l], bn=2000)
        m_flat = m.reshape(R * N, D)
        parts = _sc_scatter_kernel()(m_flat, rowidx3, dst3, zeros_acc)
        h = _update(h, parts, nt3, Wn1[l], bn1[l], Wn2[l], bn2[l], bn=bn)
    return h


# speedup vs baseline: 1.3260x; 1.0199x over previous
import functools

import jax
import jax.numpy as jnp
from jax import lax
from jax.experimental import pallas as pl
from jax.experimental.pallas import tpu as pltpu
from jax.experimental.pallas import tpu_sc as plsc

N = 10000
E = 320000
D = 128
H = 128
R = 4
T = 2
L = 2

NC = 2
NS = 16
NW = NC * NS

K = 128
CHUNKS = 80
E_PAD = NW * CHUNKS * K
N_ACC = 10112
ROWS_PER_TILE = N_ACC // NS


def _msg_body(h_ref, w1_ref, b1_ref, w2_ref, b2_ref, out_ref):
    h = h_ref[...]
    for r in range(R):
        m1 = jnp.maximum(
            jnp.dot(h, w1_ref[r], preferred_element_type=jnp.float32)
            + b1_ref[r][None, :],
            0.0,
        )
        out_ref[r] = (jnp.dot(m1, w2_ref[r], preferred_element_type=jnp.float32)
                      + b2_ref[r][None, :])


def _messages(h, We1, be1, We2, be2, bn):
    nb = N // bn
    return pl.pallas_call(
        _msg_body,
        grid=(nb,),
        in_specs=[
            pl.BlockSpec((bn, D), lambda i: (i, 0)),
            pl.BlockSpec((R, D, H), lambda i: (0, 0, 0)),
            pl.BlockSpec((R, H), lambda i: (0, 0)),
            pl.BlockSpec((R, H, D), lambda i: (0, 0, 0)),
            pl.BlockSpec((R, D), lambda i: (0, 0)),
        ],
        out_specs=pl.BlockSpec((R, bn, D), lambda i: (0, i, 0)),
        out_shape=jax.ShapeDtypeStruct((R, N, D), jnp.float32),
    )(h, We1, be1, We2, be2)


_PREP_BR = 512


def _prep_body(src_ref, et_ref, dst_ref, ri_ref, dstpad_ref):
    i = pl.program_id(0)
    rows = lax.broadcasted_iota(jnp.int32, (_PREP_BR, K), 0)
    cols = lax.broadcasted_iota(jnp.int32, (_PREP_BR, K), 1)
    flat = (i * _PREP_BR + rows) * K + cols
    mask = flat < E
    ri_ref[...] = jnp.where(mask, et_ref[...] * N + src_ref[...], flat % N)
    dstpad_ref[...] = jnp.where(mask, dst_ref[...], N + flat % (N_ACC - N))


def _edge_prep(src2d, et2d, dst2d):
    rows_out = E_PAD // K
    grid = rows_out // _PREP_BR
    return pl.pallas_call(
        _prep_body,
        grid=(grid,),
        in_specs=[
            pl.BlockSpec((_PREP_BR, K), lambda i: (i, 0)),
            pl.BlockSpec((_PREP_BR, K), lambda i: (i, 0)),
            pl.BlockSpec((_PREP_BR, K), lambda i: (i, 0)),
        ],
        out_specs=[
            pl.BlockSpec((_PREP_BR, K), lambda i: (i, 0)),
            pl.BlockSpec((_PREP_BR, K), lambda i: (i, 0)),
        ],
        out_shape=[
            jax.ShapeDtypeStruct((rows_out, K), jnp.int32),
            jax.ShapeDtypeStruct((rows_out, K), jnp.int32),
        ],
    )(src2d, et2d, dst2d)


def _sc_scatter_body(m_hbm, rowidx_hbm, dst_hbm, zeros_hbm, out_hbm,
                     idx_v, dstv, rows0, rows1, agg_sh, sem0, sem1):
    c = lax.axis_index("c")
    s = lax.axis_index("s")
    gw = s * NC + c

    r0 = s * ROWS_PER_TILE
    pltpu.sync_copy(zeros_hbm.at[pl.ds(r0, ROWS_PER_TILE)],
                    agg_sh.at[pl.ds(r0, ROWS_PER_TILE)])

    plsc.subcore_barrier()

    half = CHUNKS // 2
    for g in range(2):
        pltpu.sync_copy(rowidx_hbm.at[gw].at[pl.ds(g * half, half)], idx_v)
        pltpu.sync_copy(dst_hbm.at[gw].at[pl.ds(g * half, half)], dstv)
        pltpu.async_copy(m_hbm.at[idx_v.at[0]], rows0, sem0)

        def body(i, carry):
            j = 2 * i
            pltpu.async_copy(m_hbm.at[idx_v.at[j + 1]], rows1, sem1)
            pltpu.make_async_copy(m_hbm.at[idx_v.at[j]], rows0, sem0).wait()
            pltpu.sync_copy(rows0, agg_sh.at[dstv.at[j]], add=True)

            @pl.when(j + 2 < half)
            def _():
                pltpu.async_copy(m_hbm.at[idx_v.at[j + 2]], rows0, sem0)

            pltpu.make_async_copy(m_hbm.at[idx_v.at[j + 1]], rows1, sem1).wait()
            pltpu.sync_copy(rows1, agg_sh.at[dstv.at[j + 1]], add=True)
            return carry

        lax.fori_loop(0, half // 2, body, 0)
    plsc.subcore_barrier()

    pltpu.sync_copy(agg_sh.at[pl.ds(r0, ROWS_PER_TILE)],
                    out_hbm.at[c].at[pl.ds(r0, ROWS_PER_TILE)])


@functools.lru_cache(maxsize=1)
def _sc_scatter_kernel():
    return pl.kernel(
        _sc_scatter_body,
        mesh=plsc.VectorSubcoreMesh(
            core_axis_name="c", subcore_axis_name="s",
            num_cores=NC, num_subcores=NS,
        ),
        out_type=jax.ShapeDtypeStruct((NC, N_ACC, D), jnp.float32),
        scratch_types=[
            pltpu.VMEM((CHUNKS // 2, K), jnp.int32),
            pltpu.VMEM((CHUNKS // 2, K), jnp.int32),
            pltpu.VMEM((K, D), jnp.float32),
            pltpu.VMEM((K, D), jnp.float32),
            pltpu.VMEM_SHARED((N_ACC, D), jnp.float32),
            pltpu.SemaphoreType.DMA,
            pltpu.SemaphoreType.DMA,
        ],
    )


def _update_body(h_ref, parts_ref, nt_ref, w1_ref, b1_ref, w2_ref, b2_ref, out_ref):
    h = h_ref[...]
    agg = parts_ref[0] + parts_ref[1]
    z = jnp.concatenate([h, agg], axis=1)
    nt = nt_ref[...]
    acc = h
    for t in range(T):
        u1 = jnp.maximum(
            jnp.dot(z, w1_ref[t], preferred_element_type=jnp.float32) + b1_ref[t][None, :],
            0.0,
        )
        u = jnp.dot(u1, w2_ref[t], preferred_element_type=jnp.float32) + b2_ref[t][None, :]
        acc = jnp.where(nt == t, u, acc)
    out_ref[...] = acc


def _update(h, parts, nt3, Wn1, bn1, Wn2, bn2, bn):
    nb = N // bn
    return pl.pallas_call(
        _update_body,
        grid=(nb,),
        in_specs=[
            pl.BlockSpec((bn, D), lambda i: (i, 0)),
            pl.BlockSpec((NC, bn, D), lambda i: (0, i, 0)),
            pl.BlockSpec((bn, 1), lambda i: (i, 0)),
            pl.BlockSpec((T, 2 * D, H), lambda i: (0, 0, 0)),
            pl.BlockSpec((T, H), lambda i: (0, 0)),
            pl.BlockSpec((T, H, D), lambda i: (0, 0, 0)),
            pl.BlockSpec((T, D), lambda i: (0, 0)),
        ],
        out_specs=pl.BlockSpec((bn, D), lambda i: (i, 0)),
        out_shape=jax.ShapeDtypeStruct((N, D), jnp.float32),
    )(h, parts, nt3, Wn1, bn1, Wn2, bn2)


def kernel(node_feature, edge_index, edge_types, node_types,
           update_node_type_indices, update_edge_type_indices,
           We1, be1, We2, be2, Wn1, bn1, Wn2, bn2):
    del update_node_type_indices, update_edge_type_indices

    src = edge_index[0].astype(jnp.int32)
    dst = edge_index[1].astype(jnp.int32)
    et = edge_types.astype(jnp.int32)

    ri2d, dstpad2d = _edge_prep(src.reshape(E // K, K), et.reshape(E // K, K),
                                dst.reshape(E // K, K))
    rowidx3 = ri2d.reshape(NW, CHUNKS, K)
    dst3 = dstpad2d.reshape(NW, CHUNKS, K)

    zeros_acc = jnp.zeros((N_ACC, D), jnp.float32)
    nt3 = node_types.astype(jnp.int32).reshape(N, 1)

    bn = 2000
    h = node_feature
    for l in range(L):
        m = _messages(h, We1[l], be1[l], We2[l], be2[l], bn=2000)
        m_flat = m.reshape(R * N, D)
        parts = _sc_scatter_kernel()(m_flat, rowidx3, dst3, zeros_acc)
        h = _update(h, parts, nt3, Wn1[l], bn1[l], Wn2[l], bn2[l], bn=bn)
    return h
